# baseline (speedup 1.0000x reference)
import functools

import jax
import jax.numpy as jnp
from jax import lax
from jax.experimental import pallas as pl
from jax.experimental.pallas import tpu as pltpu
from jax.experimental.pallas import tpu_sc as plsc

_NC = 2
_NS = 16
_L = 16
_NW = _NC * _NS
_CH = 128


def kernel(target, vector_embedding, centers):
    B, D = vector_embedding.shape
    b_per_w = B // _NW
    n_ch = b_per_w // _CH
    n_vec = D // _L

    tgt3 = target.reshape(_NW, n_ch, _CH)
    emb4 = vector_embedding.reshape(_NW, n_ch, _CH, D)

    mesh = plsc.VectorSubcoreMesh(core_axis_name="c", subcore_axis_name="s")

    @functools.partial(
        pl.kernel,
        mesh=mesh,
        out_type=jax.ShapeDtypeStruct((_NW, _L), jnp.float32),
        scratch_types=[
            pltpu.VMEM((n_ch, _CH), jnp.int32),
            pltpu.VMEM((n_ch, _CH, D), jnp.float32),
            pltpu.VMEM((n_ch, _CH, D), jnp.float32),
            pltpu.VMEM((_L,), jnp.float32),
            pltpu.SemaphoreType.DMA,
        ],
        compiler_params=pltpu.CompilerParams(use_tc_tiling_on_sc=False),
    )
    def sc_kernel(tgt_hbm, emb_hbm, cen_hbm, out_hbm, idx_v, emb_v, rows_v,
                  acc_v, sem):
        wid = lax.axis_index("s") * _NC + lax.axis_index("c")

        pltpu.sync_copy(tgt_hbm.at[wid], idx_v)
        copies = [
            pltpu.async_copy(cen_hbm.at[idx_v.at[j]], rows_v.at[j], sem)
            for j in range(n_ch)
        ]
        pltpu.sync_copy(emb_hbm.at[wid], emb_v)
        for cp in copies:
            cp.wait()

        zero = jnp.zeros((_L,), jnp.float32)
        accs = (zero,) * n_vec
        for j in range(n_ch):
            def body(i, accs):
                new = []
                for c in range(n_vec):
                    e = emb_v[j, i, pl.ds(c * _L, _L)]
                    r = rows_v[j, i, pl.ds(c * _L, _L)]
                    d = e - r
                    new.append(accs[c] + d * d)
                return tuple(new)
            accs = lax.fori_loop(0, _CH, body, accs)

        acc = accs[0]
        for c in range(1, n_vec):
            acc = acc + accs[c]
        acc_v[...] = acc * (0.5 / B)
        pltpu.sync_copy(acc_v, out_hbm.at[wid])

    partials = sc_kernel(tgt3, emb4, centers)
    return jnp.sum(partials)
